# Initial kernel scaffold; baseline (speedup 1.0000x reference)
import jax
import jax.numpy as jnp
from jax.experimental import pallas as pl


def kernel(edge_feat_0, edge_feat_1, edge_feat_2, table_0, table_1, table_2):
    raise NotImplementedError("write your pallas kernel here")


# speedup vs baseline: 6.6563x; 6.6563x over previous
import functools

import jax
import jax.numpy as jnp
from jax import lax
from jax.experimental import pallas as pl
from jax.experimental.pallas import tpu as pltpu
from jax.experimental.pallas import tpu_sc as plsc

E = 320000
D = 128
V0, V1, V2 = 12, 27, 7
C = V0 * V1 * V2

NC, NS = 2, 16
NW = NC * NS
PER_W = E // NW
B = 80
STEPS = PER_W // B


def _prep_body(f0_ref, f1_ref, f2_ref, t0_ref, t1_ref, t2_ref,
               cidx_ref, ftab_ref):
    cidx_ref[:] = f0_ref[:] * (V1 * V2) + f1_ref[:] * V2 + f2_ref[:]
    r = lax.broadcasted_iota(jnp.int32, (C, 1), 0)
    a = r // (V1 * V2)
    b = (r // V2) % V1
    c = r % V2
    oh0 = (a == lax.broadcasted_iota(jnp.int32, (C, V0), 1)).astype(jnp.float32)
    oh1 = (b == lax.broadcasted_iota(jnp.int32, (C, V1), 1)).astype(jnp.float32)
    oh2 = (c == lax.broadcasted_iota(jnp.int32, (C, V2), 1)).astype(jnp.float32)
    ftab_ref[:] = (
        jnp.dot(oh0, t0_ref[:], preferred_element_type=jnp.float32)
        + jnp.dot(oh1, t1_ref[:], preferred_element_type=jnp.float32)
        + jnp.dot(oh2, t2_ref[:], preferred_element_type=jnp.float32)
    )


_prep = pl.pallas_call(
    _prep_body,
    out_shape=(
        jax.ShapeDtypeStruct((E // 128, 128), jnp.int32),
        jax.ShapeDtypeStruct((C, D), jnp.float32),
    ),
)


def _sc_gather_fn():
    mesh = plsc.VectorSubcoreMesh(
        core_axis_name="c", subcore_axis_name="s",
        num_cores=NC, num_subcores=NS)

    @functools.partial(
        pl.kernel,
        mesh=mesh,
        out_type=jax.ShapeDtypeStruct((E, D), jnp.float32),
        scratch_types=[
            pltpu.VMEM((B,), jnp.int32),
            pltpu.VMEM((B, D), jnp.float32),
            pltpu.SemaphoreType.DMA,
        ],
    )
    def sc_gather(ftab_hbm, cidx_hbm, out_hbm, idx_v, rows_v, sem):
        wid = lax.axis_index("s") * NC + lax.axis_index("c")
        base = wid * PER_W

        def body(j, carry):
            off = base + j * B
            pltpu.sync_copy(cidx_hbm.at[pl.ds(off, B)], idx_v)
            pltpu.async_copy(ftab_hbm.at[idx_v], rows_v, sem).wait()
            pltpu.sync_copy(rows_v, out_hbm.at[pl.ds(off, B)])
            return carry

        lax.fori_loop(0, STEPS, body, 0)

    return sc_gather


_sc_gather = _sc_gather_fn()


def kernel(edge_feat_0, edge_feat_1, edge_feat_2, table_0, table_1, table_2):
    f0 = edge_feat_0.reshape(E // 128, 128)
    f1 = edge_feat_1.reshape(E // 128, 128)
    f2 = edge_feat_2.reshape(E // 128, 128)
    cidx2d, ftab = _prep(f0, f1, f2, table_0, table_1, table_2)
    return _sc_gather(ftab, cidx2d.reshape(E))


# speedup vs baseline: 11.7451x; 1.7645x over previous
import functools

import jax
import jax.numpy as jnp
from jax import lax
from jax.experimental import pallas as pl
from jax.experimental.pallas import tpu as pltpu
from jax.experimental.pallas import tpu_sc as plsc

E = 320000
D = 128
V0, V1, V2 = 12, 27, 7
C = V0 * V1 * V2

NC, NS = 2, 16
NW = NC * NS
PER_W = E // NW
B = 80
K = 5
CHUNK = B * K
STEPS = PER_W // CHUNK
ROWS_PER_W = PER_W // B


def _prep_body(f0_ref, f1_ref, f2_ref, t0_ref, t1_ref, t2_ref,
               cidx_ref, ftab_ref):
    cidx_ref[:] = f0_ref[:] * (V1 * V2) + f1_ref[:] * V2 + f2_ref[:]
    r = lax.broadcasted_iota(jnp.int32, (C, 1), 0)
    a = r // (V1 * V2)
    b = (r // V2) % V1
    c = r % V2
    oh0 = (a == lax.broadcasted_iota(jnp.int32, (C, V0), 1)).astype(jnp.float32)
    oh1 = (b == lax.broadcasted_iota(jnp.int32, (C, V1), 1)).astype(jnp.float32)
    oh2 = (c == lax.broadcasted_iota(jnp.int32, (C, V2), 1)).astype(jnp.float32)
    hi = lax.Precision.HIGHEST
    ftab_ref[:] = (
        jnp.dot(oh0, t0_ref[:], preferred_element_type=jnp.float32, precision=hi)
        + jnp.dot(oh1, t1_ref[:], preferred_element_type=jnp.float32, precision=hi)
        + jnp.dot(oh2, t2_ref[:], preferred_element_type=jnp.float32, precision=hi)
    )


_prep = pl.pallas_call(
    _prep_body,
    out_shape=(
        jax.ShapeDtypeStruct((E // 128, 128), jnp.int32),
        jax.ShapeDtypeStruct((C, D), jnp.float32),
    ),
)


def _sc_gather_fn():
    mesh = plsc.VectorSubcoreMesh(
        core_axis_name="c", subcore_axis_name="s",
        num_cores=NC, num_subcores=NS)

    @functools.partial(
        pl.kernel,
        mesh=mesh,
        out_type=jax.ShapeDtypeStruct((E, D), jnp.float32),
        scratch_types=[
            pltpu.VMEM((PER_W,), jnp.int32),
            pltpu.VMEM((2, CHUNK, D), jnp.float32),
            pltpu.SemaphoreType.DMA,
            pltpu.SemaphoreType.DMA,
            pltpu.SemaphoreType.DMA,
            pltpu.SemaphoreType.DMA,
        ],
    )
    def sc_gather(ftab_hbm, cidx_hbm, out_hbm, idx_v, rows_v, g0, g1, s0, s1):
        gsem = (g0, g1)
        ssem = (s0, s1)
        wid = lax.axis_index("s") * NC + lax.axis_index("c")
        out_base = wid * PER_W

        pltpu.sync_copy(cidx_hbm.at[pl.ds(wid * PER_W, PER_W)], idx_v)

        def issue_gathers(s, p):
            for k in range(K):
                pltpu.async_copy(
                    ftab_hbm.at[idx_v.at[pl.ds((s * K + k) * B, B)]],
                    rows_v.at[p, pl.ds(k * B, B)],
                    gsem[p])

        def drain_gathers(p):
            pltpu.make_async_copy(
                out_hbm.at[pl.ds(0, CHUNK)], rows_v.at[p], gsem[p]).wait()

        def start_store(s, p):
            pltpu.make_async_copy(
                rows_v.at[p],
                out_hbm.at[pl.ds(out_base + s * CHUNK, CHUNK)],
                ssem[p]).start()

        def drain_store(p):
            pltpu.make_async_copy(
                out_hbm.at[pl.ds(0, CHUNK)], rows_v.at[p], ssem[p]).wait()

        def step(s, p, first, last):
            drain_gathers(p)
            start_store(s, p)
            if not first:
                drain_store(1 - p)
            if not last:
                issue_gathers(s + 1, 1 - p)

        issue_gathers(0, 0)
        step(0, 0, first=True, last=False)

        def body(i, carry):
            s = 2 * i + 1
            step(s, 1, first=False, last=False)
            step(s + 1, 0, first=False, last=False)
            return carry

        lax.fori_loop(0, (STEPS - 3) // 2, body, 0)

        step(STEPS - 2, 1, first=False, last=False)
        step(STEPS - 1, 0, first=False, last=True)
        drain_store(0)

    return sc_gather


_sc_gather = _sc_gather_fn()


def kernel(edge_feat_0, edge_feat_1, edge_feat_2, table_0, table_1, table_2):
    f0 = edge_feat_0.reshape(E // 128, 128)
    f1 = edge_feat_1.reshape(E // 128, 128)
    f2 = edge_feat_2.reshape(E // 128, 128)
    cidx2d, ftab = _prep(f0, f1, f2, table_0, table_1, table_2)
    return _sc_gather(ftab, cidx2d.reshape(E))


# speedup vs baseline: 21.4023x; 1.8222x over previous
import functools

import jax
import jax.numpy as jnp
from jax import lax
from jax.experimental import pallas as pl
from jax.experimental.pallas import tpu as pltpu
from jax.experimental.pallas import tpu_sc as plsc

E = 320000
D = 128
V0, V1, V2 = 12, 27, 7
C = V0 * V1 * V2

NC, NS = 2, 16
NW = NC * NS
PER_W = E // NW
B = 80
K = 5
CHUNK = B * K
STEPS = PER_W // CHUNK
ROWS_PER_W = PER_W // B


def _prep_body(f0_ref, f1_ref, f2_ref, t0_ref, t1_ref, t2_ref,
               cidx_ref, ftab_ref):
    cidx_ref[:] = f0_ref[:] * (V1 * V2) + f1_ref[:] * V2 + f2_ref[:]
    r = lax.broadcasted_iota(jnp.int32, (C, 1), 0)
    a = r // (V1 * V2)
    b = (r // V2) % V1
    c = r % V2
    oh0 = (a == lax.broadcasted_iota(jnp.int32, (C, V0), 1)).astype(jnp.float32)
    oh1 = (b == lax.broadcasted_iota(jnp.int32, (C, V1), 1)).astype(jnp.float32)
    oh2 = (c == lax.broadcasted_iota(jnp.int32, (C, V2), 1)).astype(jnp.float32)
    hi = lax.Precision.HIGHEST
    ftab_ref[:] = (
        jnp.dot(oh0, t0_ref[:], preferred_element_type=jnp.float32, precision=hi)
        + jnp.dot(oh1, t1_ref[:], preferred_element_type=jnp.float32, precision=hi)
        + jnp.dot(oh2, t2_ref[:], preferred_element_type=jnp.float32, precision=hi)
    )


_prep = pl.pallas_call(
    _prep_body,
    out_shape=(
        jax.ShapeDtypeStruct((E // 128, 128), jnp.int32),
        jax.ShapeDtypeStruct((C, D), jnp.float32),
    ),
)


def _sc_gather_fn():
    mesh = plsc.VectorSubcoreMesh(
        core_axis_name="c", subcore_axis_name="s",
        num_cores=NC, num_subcores=NS)

    @functools.partial(
        pl.kernel,
        mesh=mesh,
        out_type=jax.ShapeDtypeStruct((E, D), jnp.float32),
        scratch_types=[
            pltpu.VMEM((PER_W,), jnp.int32),
            pltpu.VMEM((2, CHUNK, D), jnp.float32),
            pltpu.VMEM_SHARED((C, D), jnp.float32),
            pltpu.SemaphoreType.DMA,
            pltpu.SemaphoreType.DMA,
            pltpu.SemaphoreType.DMA,
            pltpu.SemaphoreType.DMA,
        ],
    )
    def sc_gather(ftab_hbm, cidx_hbm, out_hbm, idx_v, rows_v, ftab_spm,
                  g0, g1, s0, s1):
        gsem = (g0, g1)
        ssem = (s0, s1)
        sid = lax.axis_index("s")
        wid = sid * NC + lax.axis_index("c")
        out_base = wid * PER_W

        @pl.when(sid == 0)
        def _():
            pltpu.sync_copy(ftab_hbm, ftab_spm)

        pltpu.sync_copy(cidx_hbm.at[pl.ds(wid * PER_W, PER_W)], idx_v)
        plsc.subcore_barrier()

        def issue_gathers(s, p):
            for k in range(K):
                pltpu.async_copy(
                    ftab_spm.at[idx_v.at[pl.ds((s * K + k) * B, B)]],
                    rows_v.at[p, pl.ds(k * B, B)],
                    gsem[p])

        def drain_gathers(p):
            pltpu.make_async_copy(
                out_hbm.at[pl.ds(0, CHUNK)], rows_v.at[p], gsem[p]).wait()

        def start_store(s, p):
            pltpu.make_async_copy(
                rows_v.at[p],
                out_hbm.at[pl.ds(out_base + s * CHUNK, CHUNK)],
                ssem[p]).start()

        def drain_store(p):
            pltpu.make_async_copy(
                out_hbm.at[pl.ds(0, CHUNK)], rows_v.at[p], ssem[p]).wait()

        def step(s, p, first, last):
            drain_gathers(p)
            start_store(s, p)
            if not first:
                drain_store(1 - p)
            if not last:
                issue_gathers(s + 1, 1 - p)

        issue_gathers(0, 0)
        step(0, 0, first=True, last=False)

        def body(i, carry):
            s = 2 * i + 1
            step(s, 1, first=False, last=False)
            step(s + 1, 0, first=False, last=False)
            return carry

        lax.fori_loop(0, (STEPS - 3) // 2, body, 0)

        step(STEPS - 2, 1, first=False, last=False)
        step(STEPS - 1, 0, first=False, last=True)
        drain_store(0)

    return sc_gather


_sc_gather = _sc_gather_fn()


def kernel(edge_feat_0, edge_feat_1, edge_feat_2, table_0, table_1, table_2):
    f0 = edge_feat_0.reshape(E // 128, 128)
    f1 = edge_feat_1.reshape(E // 128, 128)
    f2 = edge_feat_2.reshape(E // 128, 128)
    cidx2d, ftab = _prep(f0, f1, f2, table_0, table_1, table_2)
    return _sc_gather(ftab, cidx2d.reshape(E))


# speedup vs baseline: 21.5715x; 1.0079x over previous
import functools

import jax
import jax.numpy as jnp
from jax import lax
from jax.experimental import pallas as pl
from jax.experimental.pallas import tpu as pltpu
from jax.experimental.pallas import tpu_sc as plsc

E = 320000
D = 128
V0, V1, V2 = 12, 27, 7
C = V0 * V1 * V2

NC, NS = 2, 16
NW = NC * NS
PER_W = E // NW
B = 80
K = 5
CHUNK = B * K
STEPS = PER_W // CHUNK


def _ftab_body(t0_ref, t1_ref, t2_ref, ftab_ref):
    r = lax.broadcasted_iota(jnp.int32, (C, 1), 0)
    a = r // (V1 * V2)
    b = (r // V2) % V1
    c = r % V2
    oh0 = (a == lax.broadcasted_iota(jnp.int32, (C, V0), 1)).astype(jnp.float32)
    oh1 = (b == lax.broadcasted_iota(jnp.int32, (C, V1), 1)).astype(jnp.float32)
    oh2 = (c == lax.broadcasted_iota(jnp.int32, (C, V2), 1)).astype(jnp.float32)
    hi = lax.Precision.HIGHEST
    ftab_ref[:] = (
        jnp.dot(oh0, t0_ref[:], preferred_element_type=jnp.float32, precision=hi)
        + jnp.dot(oh1, t1_ref[:], preferred_element_type=jnp.float32, precision=hi)
        + jnp.dot(oh2, t2_ref[:], preferred_element_type=jnp.float32, precision=hi)
    )


_ftab = pl.pallas_call(
    _ftab_body,
    out_shape=jax.ShapeDtypeStruct((C, D), jnp.float32),
)


def _sc_gather_fn():
    mesh = plsc.VectorSubcoreMesh(
        core_axis_name="c", subcore_axis_name="s",
        num_cores=NC, num_subcores=NS)

    @functools.partial(
        pl.kernel,
        mesh=mesh,
        out_type=jax.ShapeDtypeStruct((E, D), jnp.float32),
        scratch_types=[
            pltpu.VMEM((2 * CHUNK,), jnp.int32),
            pltpu.VMEM((2 * CHUNK,), jnp.int32),
            pltpu.VMEM((2 * CHUNK,), jnp.int32),
            pltpu.VMEM((2 * CHUNK,), jnp.int32),
            pltpu.VMEM((2, CHUNK, D), jnp.float32),
            pltpu.VMEM_SHARED((C, D), jnp.float32),
            pltpu.SemaphoreType.DMA,
            pltpu.SemaphoreType.DMA,
            pltpu.SemaphoreType.DMA,
            pltpu.SemaphoreType.DMA,
            pltpu.SemaphoreType.DMA,
            pltpu.SemaphoreType.DMA,
        ],
    )
    def sc_gather(ftab_hbm, f0_hbm, f1_hbm, f2_hbm, out_hbm,
                  f0c, f1c, f2c, idxc, rows_v, ftab_spm,
                  g0, g1, s0, s1, fs0, fs1):
        gsem = (g0, g1)
        ssem = (s0, s1)
        fsem = (fs0, fs1)
        sid = lax.axis_index("s")
        wid = sid * NC + lax.axis_index("c")
        base = wid * PER_W

        def issue_f(s, slot):
            off = base + s * CHUNK
            d = pl.ds(slot * CHUNK, CHUNK)
            for src, dst in ((f0_hbm, f0c), (f1_hbm, f1c), (f2_hbm, f2c)):
                pltpu.async_copy(src.at[pl.ds(off, CHUNK)], dst.at[d], fsem[slot])

        def drain_f(slot):
            d = pl.ds(slot * CHUNK, CHUNK)
            for dst in (f0c, f1c, f2c):
                pltpu.make_async_copy(
                    f0_hbm.at[pl.ds(0, CHUNK)], dst.at[d], fsem[slot]).wait()

        def compute_cidx(slot):
            for q in range(CHUNK // 16):
                d = pl.ds(slot * CHUNK + q * 16, 16)
                idxc[d] = f0c[d] * (V1 * V2) + f1c[d] * V2 + f2c[d]

        def issue_gathers(slot):
            for k in range(K):
                pltpu.async_copy(
                    ftab_spm.at[idxc.at[pl.ds(slot * CHUNK + k * B, B)]],
                    rows_v.at[slot, pl.ds(k * B, B)],
                    gsem[slot])

        def drain_gathers(slot):
            pltpu.make_async_copy(
                out_hbm.at[pl.ds(0, CHUNK)], rows_v.at[slot], gsem[slot]).wait()

        def start_store(s, slot):
            pltpu.make_async_copy(
                rows_v.at[slot],
                out_hbm.at[pl.ds(base + s * CHUNK, CHUNK)],
                ssem[slot]).start()

        def drain_store(slot):
            pltpu.make_async_copy(
                out_hbm.at[pl.ds(0, CHUNK)], rows_v.at[slot], ssem[slot]).wait()

        def step(s, p, first=False, fmore=True, gmore=True):
            drain_gathers(p)
            start_store(s, p)
            if gmore:
                drain_f(1 - p)
                compute_cidx(1 - p)
            if fmore:
                issue_f(s + 2, p)
            if not first:
                drain_store(1 - p)
            if gmore:
                issue_gathers(1 - p)

        @pl.when(sid == 0)
        def _():
            pltpu.sync_copy(ftab_hbm, ftab_spm)
        issue_f(0, 0)
        issue_f(1, 1)
        drain_f(0)
        compute_cidx(0)
        plsc.subcore_barrier()
        issue_gathers(0)

        step(0, 0, first=True)

        def body(i, carry):
            s = 2 * i + 1
            step(s, 1)
            step(s + 1, 0)
            return carry

        lax.fori_loop(0, (STEPS - 3) // 2, body, 0)

        step(STEPS - 2, 1, fmore=False)
        step(STEPS - 1, 0, fmore=False, gmore=False)
        drain_store(0)

    return sc_gather


_sc_gather = _sc_gather_fn()


def kernel(edge_feat_0, edge_feat_1, edge_feat_2, table_0, table_1, table_2):
    ftab = _ftab(table_0, table_1, table_2)
    return _sc_gather(ftab, edge_feat_0, edge_feat_1, edge_feat_2)


# speedup vs baseline: 21.9043x; 1.0154x over previous
import functools

import jax
import jax.numpy as jnp
from jax import lax
from jax.experimental import pallas as pl
from jax.experimental.pallas import tpu as pltpu
from jax.experimental.pallas import tpu_sc as plsc

E = 320000
D = 128
V0, V1, V2 = 12, 27, 7
C = V0 * V1 * V2

NC, NS = 2, 16
NW = NC * NS
PER_W = E // NW
CHUNK = 80
STEPS = PER_W // CHUNK
LAST = STEPS - 1
NBUF = 4


def _ftab_body(t0_ref, t1_ref, t2_ref, ftab_ref):
    r = lax.broadcasted_iota(jnp.int32, (C, 1), 0)
    a = r // (V1 * V2)
    b = (r // V2) % V1
    c = r % V2
    oh0 = (a == lax.broadcasted_iota(jnp.int32, (C, V0), 1)).astype(jnp.float32)
    oh1 = (b == lax.broadcasted_iota(jnp.int32, (C, V1), 1)).astype(jnp.float32)
    oh2 = (c == lax.broadcasted_iota(jnp.int32, (C, V2), 1)).astype(jnp.float32)
    hi = lax.Precision.HIGHEST
    ftab_ref[:] = (
        jnp.dot(oh0, t0_ref[:], preferred_element_type=jnp.float32, precision=hi)
        + jnp.dot(oh1, t1_ref[:], preferred_element_type=jnp.float32, precision=hi)
        + jnp.dot(oh2, t2_ref[:], preferred_element_type=jnp.float32, precision=hi)
    )


_ftab = pl.pallas_call(
    _ftab_body,
    out_shape=jax.ShapeDtypeStruct((C, D), jnp.float32),
)


def _sc_gather_fn():
    mesh = plsc.VectorSubcoreMesh(
        core_axis_name="c", subcore_axis_name="s",
        num_cores=NC, num_subcores=NS)

    @functools.partial(
        pl.kernel,
        mesh=mesh,
        out_type=jax.ShapeDtypeStruct((E, D), jnp.float32),
        scratch_types=[
            pltpu.VMEM((NBUF * CHUNK,), jnp.int32),
            pltpu.VMEM((NBUF * CHUNK,), jnp.int32),
            pltpu.VMEM((NBUF * CHUNK,), jnp.int32),
            pltpu.VMEM((NBUF * CHUNK,), jnp.int32),
            pltpu.VMEM((NBUF, CHUNK, D), jnp.float32),
            pltpu.VMEM_SHARED((C, D), jnp.float32),
            [pltpu.SemaphoreType.DMA] * NBUF,
            [pltpu.SemaphoreType.DMA] * NBUF,
            [pltpu.SemaphoreType.DMA] * NBUF,
        ],
    )
    def sc_gather(ftab_hbm, f0_hbm, f1_hbm, f2_hbm, out_hbm,
                  f0c, f1c, f2c, idxc, rows_v, ftab_spm,
                  gsem, ssem, fsem):
        sid = lax.axis_index("s")
        wid = sid * NC + lax.axis_index("c")
        base = wid * PER_W

        def issue_f(s, slot):
            off = base + s * CHUNK
            d = pl.ds(slot * CHUNK, CHUNK)
            for src, dst in ((f0_hbm, f0c), (f1_hbm, f1c), (f2_hbm, f2c)):
                pltpu.async_copy(src.at[pl.ds(off, CHUNK)], dst.at[d], fsem[slot])

        def drain_f(slot):
            d = pl.ds(slot * CHUNK, CHUNK)
            for dst in (f0c, f1c, f2c):
                pltpu.make_async_copy(
                    f0_hbm.at[pl.ds(0, CHUNK)], dst.at[d], fsem[slot]).wait()

        def compute_cidx(slot):
            for q in range(CHUNK // 16):
                d = pl.ds(slot * CHUNK + q * 16, 16)
                idxc[d] = f0c[d] * (V1 * V2) + f1c[d] * V2 + f2c[d]

        def issue_gather(slot):
            pltpu.async_copy(
                ftab_spm.at[idxc.at[pl.ds(slot * CHUNK, CHUNK)]],
                rows_v.at[slot], gsem[slot])

        def drain_gather(slot):
            pltpu.make_async_copy(
                out_hbm.at[pl.ds(0, CHUNK)], rows_v.at[slot], gsem[slot]).wait()

        def start_store(s, slot):
            pltpu.make_async_copy(
                rows_v.at[slot],
                out_hbm.at[pl.ds(base + s * CHUNK, CHUNK)],
                ssem[slot]).start()

        def drain_store(slot):
            pltpu.make_async_copy(
                out_hbm.at[pl.ds(0, CHUNK)], rows_v.at[slot], ssem[slot]).wait()

        def step(s, p, smin2=True, gmore=True, fmore=True):
            q = (p + 2) % NBUF
            drain_gather(p)
            start_store(s, p)
            if gmore:
                drain_f(q)
                compute_cidx(q)
            if smin2:
                drain_store(q)
            if gmore:
                issue_gather(q)
            if fmore:
                issue_f(s + 4, p)

        @pl.when(sid == 0)
        def _():
            pltpu.sync_copy(ftab_hbm, ftab_spm)
        for s in range(NBUF):
            issue_f(s, s)
        drain_f(0)
        compute_cidx(0)
        drain_f(1)
        compute_cidx(1)
        plsc.subcore_barrier()
        issue_gather(0)
        issue_gather(1)

        step(0, 0, smin2=False)
        step(1, 1, smin2=False)

        def body(i, carry):
            s = 4 * i + 2
            step(s, 2)
            step(s + 1, 3)
            step(s + 2, 0)
            step(s + 3, 1)
            return carry

        lax.fori_loop(0, 29, body, 0)

        step(118, 2)
        step(119, 3)
        step(120, 0)
        step(121, 1, fmore=False)
        step(122, 2, fmore=False)
        step(123, 3, gmore=False, fmore=False)
        step(124, 0, gmore=False, fmore=False)
        drain_store(3)
        drain_store(0)

    return sc_gather


_sc_gather = _sc_gather_fn()


def kernel(edge_feat_0, edge_feat_1, edge_feat_2, table_0, table_1, table_2):
    ftab = _ftab(table_0, table_1, table_2)
    return _sc_gather(ftab, edge_feat_0, edge_feat_1, edge_feat_2)


# speedup vs baseline: 22.4882x; 1.0267x over previous
import functools

import jax
import jax.numpy as jnp
from jax import lax
from jax.experimental import pallas as pl
from jax.experimental.pallas import tpu as pltpu
from jax.experimental.pallas import tpu_sc as plsc

E = 320000
D = 128
V0, V1, V2 = 12, 27, 7
C = V0 * V1 * V2

NC, NS = 2, 16
NW = NC * NS
PER_W = E // NW
CHUNK = 80
STEPS = PER_W // CHUNK
LAST = STEPS - 1
NBUF = 8
LEAD = 3


def _ftab_body(t0_ref, t1_ref, t2_ref, ftab_ref):
    r = lax.broadcasted_iota(jnp.int32, (C, 1), 0)
    a = r // (V1 * V2)
    b = (r // V2) % V1
    c = r % V2
    oh0 = (a == lax.broadcasted_iota(jnp.int32, (C, V0), 1)).astype(jnp.float32)
    oh1 = (b == lax.broadcasted_iota(jnp.int32, (C, V1), 1)).astype(jnp.float32)
    oh2 = (c == lax.broadcasted_iota(jnp.int32, (C, V2), 1)).astype(jnp.float32)
    hi = lax.Precision.HIGHEST
    ftab_ref[:] = (
        jnp.dot(oh0, t0_ref[:], preferred_element_type=jnp.float32, precision=hi)
        + jnp.dot(oh1, t1_ref[:], preferred_element_type=jnp.float32, precision=hi)
        + jnp.dot(oh2, t2_ref[:], preferred_element_type=jnp.float32, precision=hi)
    )


_ftab = pl.pallas_call(
    _ftab_body,
    out_shape=jax.ShapeDtypeStruct((C, D), jnp.float32),
)


def _sc_gather_fn():
    mesh = plsc.VectorSubcoreMesh(
        core_axis_name="c", subcore_axis_name="s",
        num_cores=NC, num_subcores=NS)

    @functools.partial(
        pl.kernel,
        mesh=mesh,
        out_type=jax.ShapeDtypeStruct((E, D), jnp.float32),
        scratch_types=[
            pltpu.VMEM((NBUF * CHUNK,), jnp.int32),
            pltpu.VMEM((NBUF * CHUNK,), jnp.int32),
            pltpu.VMEM((NBUF * CHUNK,), jnp.int32),
            pltpu.VMEM((NBUF * CHUNK,), jnp.int32),
            pltpu.VMEM((NBUF, CHUNK, D), jnp.float32),
            pltpu.VMEM_SHARED((C, D), jnp.float32),
            [pltpu.SemaphoreType.DMA] * NBUF,
            [pltpu.SemaphoreType.DMA] * NBUF,
            [pltpu.SemaphoreType.DMA] * NBUF,
        ],
    )
    def sc_gather(ftab_hbm, f0_hbm, f1_hbm, f2_hbm, out_hbm,
                  f0c, f1c, f2c, idxc, rows_v, ftab_spm,
                  gsem, ssem, fsem):
        sid = lax.axis_index("s")
        wid = sid * NC + lax.axis_index("c")
        base = wid * PER_W

        def issue_f(s, slot):
            off = base + s * CHUNK
            d = pl.ds(slot * CHUNK, CHUNK)
            for src, dst in ((f0_hbm, f0c), (f1_hbm, f1c), (f2_hbm, f2c)):
                pltpu.async_copy(src.at[pl.ds(off, CHUNK)], dst.at[d], fsem[slot])

        def drain_f(slot):
            d = pl.ds(slot * CHUNK, CHUNK)
            for dst in (f0c, f1c, f2c):
                pltpu.make_async_copy(
                    f0_hbm.at[pl.ds(0, CHUNK)], dst.at[d], fsem[slot]).wait()

        def compute_cidx(slot):
            for q in range(CHUNK // 16):
                d = pl.ds(slot * CHUNK + q * 16, 16)
                idxc[d] = f0c[d] * (V1 * V2) + f1c[d] * V2 + f2c[d]

        def issue_gather(slot):
            pltpu.async_copy(
                ftab_spm.at[idxc.at[pl.ds(slot * CHUNK, CHUNK)]],
                rows_v.at[slot], gsem[slot])

        def drain_gather(slot):
            pltpu.make_async_copy(
                out_hbm.at[pl.ds(0, CHUNK)], rows_v.at[slot], gsem[slot]).wait()

        def start_store(s, slot):
            pltpu.make_async_copy(
                rows_v.at[slot],
                out_hbm.at[pl.ds(base + s * CHUNK, CHUNK)],
                ssem[slot]).start()

        def drain_store(slot):
            pltpu.make_async_copy(
                out_hbm.at[pl.ds(0, CHUNK)], rows_v.at[slot], ssem[slot]).wait()

        def step(s, p, sdrain=True, gmore=True, fmore=True):
            q = (p + LEAD) % NBUF
            drain_gather(p)
            start_store(s, p)
            if gmore:
                drain_f(q)
                compute_cidx(q)
            if sdrain:
                drain_store(q)
            if gmore:
                issue_gather(q)
            if fmore:
                issue_f(s + NBUF, p)

        @pl.when(sid == 0)
        def _():
            pltpu.sync_copy(ftab_hbm, ftab_spm)
        for s in range(NBUF):
            issue_f(s, s)
        for s in range(LEAD):
            drain_f(s)
            compute_cidx(s)
        plsc.subcore_barrier()
        for s in range(LEAD):
            issue_gather(s)

        for s in range(NBUF - LEAD):
            step(s, s, sdrain=False)

        def body(i, carry):
            s = NBUF * i + (NBUF - LEAD)
            for j in range(NBUF):
                step(s + j, (NBUF - LEAD + j) % NBUF)
            return carry

        lax.fori_loop(0, (STEPS - NBUF - (NBUF - LEAD)) // NBUF, body, 0)

        for s in range(STEPS - NBUF, STEPS):
            step(s, s % NBUF,
                 gmore=(s + LEAD <= LAST),
                 fmore=(s + NBUF <= LAST))
        for s in range(STEPS - (NBUF - LEAD), STEPS):
            drain_store(s % NBUF)

    return sc_gather


_sc_gather = _sc_gather_fn()


def kernel(edge_feat_0, edge_feat_1, edge_feat_2, table_0, table_1, table_2):
    ftab = _ftab(table_0, table_1, table_2)
    return _sc_gather(ftab, edge_feat_0, edge_feat_1, edge_feat_2)
